# baseline (speedup 1.0000x reference)
import jax
import jax.numpy as jnp
from jax import lax
from jax.experimental import pallas as pl
from jax.experimental.pallas import tpu as pltpu
from jax.experimental.pallas import tpu_sc as plsc

_NC = 2
_NS = 16
_NW = _NC * _NS


def _emb_body(ids_hbm, wte_hbm, wpe_hbm, out_hbm, idx_v, wpe_v, rows_v, gsem):
    B, _ = ids_hbm.shape
    P, D = wpe_v.shape
    wid = lax.axis_index("s") * _NC + lax.axis_index("c")
    p0 = wid * P
    pltpu.sync_copy(wpe_hbm.at[pl.ds(p0, P)], wpe_v)

    def batch_body(b, carry):
        pltpu.sync_copy(ids_hbm.at[b, pl.ds(p0, P)], idx_v)
        pltpu.async_copy(wte_hbm.at[idx_v], rows_v, gsem).wait()

        def row_body(i, c2):
            for j in range(D // 16):
                sl = pl.ds(j * 16, 16)
                rows_v[i, sl] = rows_v[i, sl] + wpe_v[i, sl]
            return c2

        lax.fori_loop(0, P, row_body, 0)
        pltpu.sync_copy(rows_v, out_hbm.at[b, pl.ds(p0, P)])
        return carry

    lax.fori_loop(0, B, batch_body, 0)


def kernel(input_ids, wte, wpe):
    B, S = input_ids.shape
    V, D = wte.shape
    P = S // _NW
    mesh = plsc.VectorSubcoreMesh(
        core_axis_name="c", subcore_axis_name="s",
        num_cores=_NC, num_subcores=_NS,
    )
    f = pl.kernel(
        _emb_body,
        out_type=jax.ShapeDtypeStruct((B, S, D), jnp.float32),
        mesh=mesh,
        scratch_types=[
            pltpu.VMEM((P,), jnp.int32),
            pltpu.VMEM((P, D), jnp.float32),
            pltpu.VMEM((P, D), jnp.float32),
            pltpu.SemaphoreType.DMA,
        ],
    )
    return f(input_ids.astype(jnp.int32), wte, wpe)


# speedup vs baseline: 2.0924x; 2.0924x over previous
import jax
import jax.numpy as jnp
from jax import lax
from jax.experimental import pallas as pl
from jax.experimental.pallas import tpu as pltpu
from jax.experimental.pallas import tpu_sc as plsc

_NC = 2
_NS = 16
_NW = _NC * _NS
_H = 16


def _emb_body(ids_hbm, wte_hbm, wpe_hbm, out_hbm,
              idx_all, wpe_v, gbuf0, gbuf1, obuf0, obuf1,
              gs0, gs1, ws0, ws1, isem):
    B, _ = ids_hbm.shape
    P, D = wpe_v.shape
    wid = lax.axis_index("s") * _NC + lax.axis_index("c")
    p0 = wid * P
    idx_copies = [
        pltpu.make_async_copy(ids_hbm.at[b, pl.ds(p0, P)], idx_all.at[b], isem)
        for b in range(B)
    ]
    for c in idx_copies:
        c.start()
    pltpu.sync_copy(wpe_hbm.at[pl.ds(p0, P)], wpe_v)
    for c in idx_copies:
        c.wait()

    gbufs = (gbuf0, gbuf1)
    obufs = (obuf0, obuf1)
    gsems = (gs0, gs1)
    wsems = (ws0, ws1)

    def gather(k, r):
        idx = idx_all.at[k, pl.ds(r * _H, _H)]
        return pltpu.make_async_copy(wte_hbm.at[idx], gbufs[r], gsems[r])

    def write(k, r):
        dst = out_hbm.at[k, pl.ds(p0 + r * _H, _H)]
        return pltpu.make_async_copy(obufs[r], dst, wsems[r])

    def add_rows(r):
        g, o = gbufs[r], obufs[r]

        def row_body(i, c):
            for j in range(D // 16):
                sl = pl.ds(j * 16, 16)
                o[i, sl] = g[i, sl] + wpe_v[r * _H + i, sl]
            return c

        lax.fori_loop(0, _H, row_body, 0)

    gather(0, 0).start()
    gather(0, 1).start()

    def batch_body(k, carry):
        for r in range(2):
            gather(k, r).wait()

            @pl.when(k > 0)
            def _():
                write(k - 1, r).wait()

            add_rows(r)
            write(k, r).start()

            @pl.when(k < B - 1)
            def _():
                gather(k + 1, r).start()

        return carry

    lax.fori_loop(0, B, batch_body, 0)
    write(B - 1, 0).wait()
    write(B - 1, 1).wait()


def kernel(input_ids, wte, wpe):
    B, S = input_ids.shape
    V, D = wte.shape
    P = S // _NW
    mesh = plsc.VectorSubcoreMesh(
        core_axis_name="c", subcore_axis_name="s",
        num_cores=_NC, num_subcores=_NS,
    )
    f = pl.kernel(
        _emb_body,
        out_type=jax.ShapeDtypeStruct((B, S, D), jnp.float32),
        mesh=mesh,
        scratch_types=[
            pltpu.VMEM((B, P), jnp.int32),
            pltpu.VMEM((P, D), jnp.float32),
            pltpu.VMEM((_H, D), jnp.float32),
            pltpu.VMEM((_H, D), jnp.float32),
            pltpu.VMEM((_H, D), jnp.float32),
            pltpu.VMEM((_H, D), jnp.float32),
            pltpu.SemaphoreType.DMA,
            pltpu.SemaphoreType.DMA,
            pltpu.SemaphoreType.DMA,
            pltpu.SemaphoreType.DMA,
            pltpu.SemaphoreType.DMA,
        ],
    )
    return f(input_ids.astype(jnp.int32), wte, wpe)
